# baseline (speedup 1.0000x reference)
import functools

import jax
import jax.numpy as jnp
from jax import lax
from jax.experimental import pallas as pl
from jax.experimental.pallas import tpu as pltpu
from jax.experimental.pallas import tpu_sc as plsc

_NUM_USER = 1000000
_NUM_LIST = 100000
_D = 16
_NC = 2
_NS = 16
_NW = _NC * _NS
_CHUNK = 128


@functools.lru_cache(maxsize=None)
def _build(B: int):
    assert B % (_NW * _CHUNK) == 0
    bpw = B // _NW
    n_chunks = bpw // _CHUNK
    mesh = plsc.VectorSubcoreMesh(core_axis_name="c", subcore_axis_name="s")

    @functools.partial(
        pl.kernel,
        out_type=jax.ShapeDtypeStruct((B,), jnp.float32),
        mesh=mesh,
        compiler_params=pltpu.CompilerParams(
            needs_layout_passes=False,
            use_tc_tiling_on_sc=False,
        ),
        scratch_types=[
            pltpu.VMEM((bpw,), jnp.int32),
            pltpu.VMEM((bpw,), jnp.int32),
            pltpu.VMEM((bpw, _D), jnp.float32),
            pltpu.VMEM((bpw, _D), jnp.float32),
            pltpu.VMEM((_D,), jnp.float32),
            pltpu.VMEM((_D,), jnp.float32),
            pltpu.VMEM((bpw,), jnp.float32),
            pltpu.SemaphoreType.DMA,
        ],
    )
    def gmf(lidx_hbm, iidx_hbm, table_hbm, w_hbm, b_hbm, out_hbm,
            li_v, ii_v, rows_l, rows_i, w_v, b_v, out_v, sem):
        wid = lax.axis_index("s") * _NC + lax.axis_index("c")
        base = wid * bpw

        pltpu.sync_copy(lidx_hbm.at[pl.ds(base, bpw)], li_v)
        pltpu.sync_copy(iidx_hbm.at[pl.ds(base, bpw)], ii_v)
        pltpu.sync_copy(w_hbm, w_v)
        pltpu.sync_copy(b_hbm, b_v)

        def add_offsets(j, carry):
            sl = pl.ds(j * 16, 16)
            li_v[sl] = li_v[sl] + _NUM_USER
            ii_v[sl] = ii_v[sl] + (_NUM_USER + _NUM_LIST)
            return carry

        lax.fori_loop(0, bpw // 16, add_offsets, 0)

        copies = []
        for c in range(n_chunks):
            sl = pl.ds(c * _CHUNK, _CHUNK)
            copies.append(
                pltpu.async_copy(table_hbm.at[li_v.at[sl]], rows_l.at[sl], sem))
            copies.append(
                pltpu.async_copy(table_hbm.at[ii_v.at[sl]], rows_i.at[sl], sem))
        for cp in copies:
            cp.wait()

        wvec = w_v[...]
        bias = b_v[...]
        lane15 = lax.iota(jnp.int32, 16) == 15

        def row(j, carry):
            prod = rows_l[j, :] * rows_i[j, :] * wvec
            csum = jnp.cumsum(prod)
            plsc.store_scatter(
                out_v, [jnp.full((16,), 0, jnp.int32) + j], csum, mask=lane15)
            return carry

        lax.fori_loop(0, bpw, row, 0, unroll=8)

        def sigm(g, carry):
            sl = pl.ds(g * 16, 16)
            x = out_v[sl] + bias
            out_v[sl] = 1.0 / (1.0 + jnp.exp(-x))
            return carry

        lax.fori_loop(0, bpw // 16, sigm, 0, unroll=4)

        pltpu.sync_copy(out_v, out_hbm.at[pl.ds(base, bpw)])

    return gmf


def kernel(user_indices, list_indices, item_indices, table, fc1_w, fc1_b):
    del user_indices
    B = list_indices.shape[0]
    fn = _build(B)
    w_flat = fc1_w.reshape(_D).astype(jnp.float32)
    b_vec = jnp.broadcast_to(fc1_b.astype(jnp.float32), (_D,))
    return fn(
        list_indices.astype(jnp.int32),
        item_indices.astype(jnp.int32),
        table,
        w_flat,
        b_vec,
    )


# speedup vs baseline: 6.5076x; 6.5076x over previous
import functools

import jax
import jax.numpy as jnp
from jax import lax
from jax.experimental import pallas as pl
from jax.experimental.pallas import tpu as pltpu
from jax.experimental.pallas import tpu_sc as plsc

_NUM_USER = 1000000
_NUM_LIST = 100000
_D = 16
_NC = 2
_NS = 16
_NW = _NC * _NS
_G = 16


@functools.lru_cache(maxsize=None)
def _build(B: int, V: int):
    assert B % (_NW * _G) == 0
    bpw = B // _NW
    mesh = plsc.VectorSubcoreMesh(core_axis_name="c", subcore_axis_name="s")

    @functools.partial(
        pl.kernel,
        out_type=jax.ShapeDtypeStruct((B,), jnp.float32),
        mesh=mesh,
        compiler_params=pltpu.CompilerParams(
            needs_layout_passes=False,
            use_tc_tiling_on_sc=True,
            disable_bounds_checks=True,
        ),
        scratch_types=[
            pltpu.VMEM((bpw,), jnp.int32),
            pltpu.VMEM((bpw,), jnp.int32),
            pltpu.VMEM((_G * _D, 128), jnp.float32),
            pltpu.VMEM((_G * _D, 128), jnp.float32),
            pltpu.VMEM((_D,), jnp.float32),
            pltpu.VMEM((_D,), jnp.float32),
            pltpu.VMEM((bpw,), jnp.float32),
            pltpu.SemaphoreType.DMA,
        ],
    )
    def gmf(tt_hbm, lidx_hbm, iidx_hbm, w_hbm, b_hbm, out_hbm,
            li_v, ii_v, ringl_v, ringi_v, w_v, b_v, out_v, sem):
        wid = lax.axis_index("s") * _NC + lax.axis_index("c")
        base = wid * bpw

        pltpu.sync_copy(lidx_hbm.at[pl.ds(base, bpw)], li_v)
        pltpu.sync_copy(iidx_hbm.at[pl.ds(base, bpw)], ii_v)
        pltpu.sync_copy(w_hbm, w_v)
        pltpu.sync_copy(b_hbm, b_v)

        wvec = w_v[...]
        iota = lax.iota(jnp.int32, _D)
        lane15 = iota == 15

        def group(g, carry):
            ivl = li_v[pl.ds(g * _G, _G)] + _NUM_USER
            ivi = ii_v[pl.ds(g * _G, _G)] + (_NUM_USER + _NUM_LIST)
            copies = []
            for k in range(_G):
                cl = pl.multiple_of((ivl[k] >> 7) * 128, 128)
                ci = pl.multiple_of((ivi[k] >> 7) * 128, 128)
                copies.append(pltpu.async_copy(
                    tt_hbm.at[:, pl.ds(cl, 128)],
                    ringl_v.at[pl.ds(k * _D, _D), :], sem))
                copies.append(pltpu.async_copy(
                    tt_hbm.at[:, pl.ds(ci, 128)],
                    ringi_v.at[pl.ds(k * _D, _D), :], sem))
            lanes_l = ivl & 127
            lanes_i = ivi & 127
            for cp in copies:
                cp.wait()
            for k in range(_G):
                rows = k * _D + iota
                vl = plsc.load_gather(ringl_v, [rows, 0 * rows + lanes_l[k]])
                vi = plsc.load_gather(ringi_v, [rows, 0 * rows + lanes_i[k]])
                csum = jnp.cumsum(vl * vi * wvec)
                plsc.store_scatter(
                    out_v, [0 * rows + (g * _G + k)], csum, mask=lane15)
            return carry

        lax.fori_loop(0, bpw // _G, group, 0)

        bias = b_v[...]

        def sigm(q, carry):
            sl = pl.ds(q * _D, _D)
            x = out_v[sl] + bias
            out_v[sl] = 1.0 / (1.0 + jnp.exp(-x))
            return carry

        lax.fori_loop(0, bpw // _D, sigm, 0)

        pltpu.sync_copy(out_v, out_hbm.at[pl.ds(base, bpw)])

    return gmf


def kernel(user_indices, list_indices, item_indices, table, fc1_w, fc1_b):
    del user_indices
    B = list_indices.shape[0]
    fn = _build(B, table.shape[0])
    w_flat = fc1_w.reshape(_D).astype(jnp.float32)
    b_vec = jnp.broadcast_to(fc1_b.astype(jnp.float32), (_D,))
    return fn(
        table.T,
        list_indices.astype(jnp.int32),
        item_indices.astype(jnp.int32),
        w_flat,
        b_vec,
    )
